# baseline (speedup 1.0000x reference)
import jax
import jax.numpy as jnp
from jax.experimental import pallas as pl
from jax.experimental.pallas import tpu as pltpu

TOKENS = 8192
HIDDEN = 2048
EXPERTS = 16
CT = 256
NCHUNK = TOKENS // CT
NBUF = 10


def _body(x_hbm, w_ref, logits_ref, idx_ref, xbuf, sems, wt_ref):
    def copy(i):
        return pltpu.make_async_copy(
            x_hbm.at[pl.ds(i * CT, CT), :], xbuf.at[i % NBUF], sems.at[i % NBUF]
        )

    for j in range(NBUF - 1):
        copy(j).start()
    w = w_ref[...]
    wt_ref[...] = w.T
    wt = wt_ref[...]
    for i in range(NCHUNK):
        if i + NBUF - 1 < NCHUNK:
            copy(i + NBUF - 1).start()
        copy(i).wait()
        xb = xbuf[i % NBUF]
        l = jnp.dot(xb, wt, preferred_element_type=jnp.float32)
        logits_ref[pl.ds(i * CT, CT), :] = l
        lt = jax.lax.dot_general(
            w, xb, (((1,), (1,)), ((), ())), preferred_element_type=jnp.float32
        )
        e_iota = jax.lax.broadcasted_iota(jnp.int32, (EXPERTS, CT), 0)
        mv, mi = lt, e_iota
        while mv.shape[0] > 1:
            h = mv.shape[0] // 2
            upd = mv[h:] > mv[:h]
            mv = jnp.where(upd, mv[h:], mv[:h])
            mi = jnp.where(upd, mi[h:], mi[:h])
        idx_ref[pl.ds(i * CT, CT)] = mi[0]


def kernel(x, W):
    logits, idx = pl.pallas_call(
        _body,
        in_specs=[
            pl.BlockSpec(memory_space=pl.ANY),
            pl.BlockSpec((EXPERTS, HIDDEN), lambda: (0, 0)),
        ],
        out_specs=[
            pl.BlockSpec((TOKENS, EXPERTS), lambda: (0, 0)),
            pl.BlockSpec((TOKENS,), lambda: (0,)),
        ],
        out_shape=[
            jax.ShapeDtypeStruct((TOKENS, EXPERTS), jnp.float32),
            jax.ShapeDtypeStruct((TOKENS,), jnp.int32),
        ],
        scratch_shapes=[
            pltpu.VMEM((NBUF, CT, HIDDEN), jnp.float32),
            pltpu.SemaphoreType.DMA((NBUF,)),
            pltpu.VMEM((HIDDEN, EXPERTS), jnp.float32),
        ],
    )(x, W)
    return (logits, idx)


# speedup vs baseline: 1.0290x; 1.0290x over previous
import jax
import jax.numpy as jnp
from jax.experimental import pallas as pl
from jax.experimental.pallas import tpu as pltpu

TOKENS = 8192
HIDDEN = 2048
EXPERTS = 16
CT = 512
NCHUNK = TOKENS // CT
NBUF = 8


def _body(x_hbm, w_ref, logits_ref, idx_ref, xbuf, sems, wt_ref):
    def copy(i):
        return pltpu.make_async_copy(
            x_hbm.at[pl.ds(i * CT, CT), :], xbuf.at[i % NBUF], sems.at[i % NBUF]
        )

    for j in range(NBUF - 1):
        copy(j).start()
    w = w_ref[...]
    wt_ref[...] = w.T
    wt = wt_ref[...]
    for i in range(NCHUNK):
        if i + NBUF - 1 < NCHUNK:
            copy(i + NBUF - 1).start()
        copy(i).wait()
        xb = xbuf[i % NBUF]
        l = jnp.dot(xb, wt, preferred_element_type=jnp.float32)
        logits_ref[pl.ds(i * CT, CT), :] = l
        lt = jax.lax.dot_general(
            w, xb, (((1,), (1,)), ((), ())), preferred_element_type=jnp.float32
        )
        e_iota = jax.lax.broadcasted_iota(jnp.int32, (EXPERTS, CT), 0)
        mv, mi = lt, e_iota
        while mv.shape[0] > 1:
            h = mv.shape[0] // 2
            upd = mv[h:] > mv[:h]
            mv = jnp.where(upd, mv[h:], mv[:h])
            mi = jnp.where(upd, mi[h:], mi[:h])
        idx_ref[pl.ds(i * CT, CT)] = mi[0]


def kernel(x, W):
    logits, idx = pl.pallas_call(
        _body,
        in_specs=[
            pl.BlockSpec(memory_space=pl.ANY),
            pl.BlockSpec((EXPERTS, HIDDEN), lambda: (0, 0)),
        ],
        out_specs=[
            pl.BlockSpec((TOKENS, EXPERTS), lambda: (0, 0)),
            pl.BlockSpec((TOKENS,), lambda: (0,)),
        ],
        out_shape=[
            jax.ShapeDtypeStruct((TOKENS, EXPERTS), jnp.float32),
            jax.ShapeDtypeStruct((TOKENS,), jnp.int32),
        ],
        scratch_shapes=[
            pltpu.VMEM((NBUF, CT, HIDDEN), jnp.float32),
            pltpu.SemaphoreType.DMA((NBUF,)),
            pltpu.VMEM((HIDDEN, EXPERTS), jnp.float32),
        ],
    )(x, W)
    return (logits, idx)
